# baseline (speedup 1.0000x reference)
import functools

import jax
import jax.numpy as jnp
from jax import lax
from jax.experimental import pallas as pl
from jax.experimental.pallas import tpu as pltpu
from jax.experimental.pallas import tpu_sc as plsc

_LB = 256
_CH = 32


def _sc_gather_rows(table, idx):
    n = idx.shape[0]
    d = table.shape[1]
    info = plsc.get_sparse_core_info()
    nw = info.num_cores * info.num_subcores
    rows_w = n // nw
    nch = rows_w // _CH
    mesh = plsc.VectorSubcoreMesh(core_axis_name="c", subcore_axis_name="s")

    @functools.partial(
        pl.kernel,
        mesh=mesh,
        out_type=jax.ShapeDtypeStruct((n, d), jnp.float32),
        scratch_types=[
            pltpu.VMEM((_CH,), jnp.int32),
            pltpu.VMEM((_CH, d), jnp.float32),
            pltpu.SemaphoreType.DMA,
        ],
    )
    def gather_k(table_hbm, idx_hbm, out_hbm, idx_v, rows_v, sem):
        wid = lax.axis_index("s") * info.num_cores + lax.axis_index("c")
        base = wid * rows_w

        def body(i, carry):
            off = base + i * _CH
            pltpu.sync_copy(idx_hbm.at[pl.ds(off, _CH)], idx_v)
            pltpu.async_copy(table_hbm.at[idx_v], rows_v, sem).wait()
            pltpu.sync_copy(rows_v, out_hbm.at[pl.ds(off, _CH)])
            return carry

        lax.fori_loop(0, nch, body, 0)

    return gather_k(table, idx)


def _tc_assemble(te3, states, mult, state_w, w8, c8):
    b, l, d = te3.shape
    sdim = states.shape[-1]
    spread = w8.shape[0]

    def body(te_ref, st_ref, m_ref, sw_ref, w_ref, c_ref, out_ref):
        k = pl.program_id(2)
        te = te_ref[0]
        m = m_ref[0, 0]
        v = m * w_ref[0] + c_ref[0]
        s = jnp.where(k < 2, jnp.float32(2.0), jnp.float32(1.0))
        v = v + s * te
        v = lax.cond(
            k == 1,
            lambda: v + jnp.dot(st_ref[0], sw_ref[...],
                                preferred_element_type=jnp.float32),
            lambda: v,
        )
        out_ref[0] = v

    return pl.pallas_call(
        body,
        grid=(b, l // _LB, spread),
        in_specs=[
            pl.BlockSpec((1, _LB, d), lambda i, j, k: (i, j, 0)),
            pl.BlockSpec((1, _LB, sdim), lambda i, j, k: (i, j, 0)),
            pl.BlockSpec((1, 1, _LB, 1), lambda i, j, k: (i, k, j, 0)),
            pl.BlockSpec((sdim, d), lambda i, j, k: (0, 0)),
            pl.BlockSpec((1, 1, d), lambda i, j, k: (k, 0, 0)),
            pl.BlockSpec((1, 1, d), lambda i, j, k: (k, 0, 0)),
        ],
        out_specs=pl.BlockSpec((1, _LB, d), lambda i, j, k: (i, j, k)),
        out_shape=jax.ShapeDtypeStruct((b, l, spread * d), jnp.float32),
        compiler_params=pltpu.CompilerParams(
            dimension_semantics=("parallel", "parallel", "arbitrary"),
        ),
    )(te3, states, mult, state_w, w8, c8)


def kernel(states, actions, returns_to_go, time_steps, padding_mask,
           timestep_table, state_W, state_b, return_W, return_b,
           act_W, act_b, action_pos_table):
    b, l, sdim = states.shape
    a = actions.shape[-1]
    d = timestep_table.shape[1]
    spread = 2 + a

    te = _sc_gather_rows(
        timestep_table, time_steps.reshape(b * l).astype(jnp.int32))
    te3 = te.reshape(b, l, d)

    zcol = jnp.zeros((b, l, 1), jnp.float32)
    mult = jnp.concatenate([returns_to_go[..., None], zcol, actions], axis=-1)
    mult = mult.transpose(0, 2, 1)[..., None]
    zrow = jnp.zeros((1, d), jnp.float32)
    w8 = jnp.concatenate(
        [return_W[None], zrow] + [act_W[None]] * a, axis=0)[:, None, :]
    c8 = jnp.concatenate(
        [return_b[None], state_b[None], act_b[None] + action_pos_table],
        axis=0)[:, None, :]

    out = _tc_assemble(te3, states, mult, state_W, w8, c8)
    embeds = out.reshape(b, l * spread, d)
    pm = jnp.repeat(padding_mask, spread, axis=1)
    return embeds, pm


# speedup vs baseline: 1.2970x; 1.2970x over previous
import functools

import jax
import jax.numpy as jnp
from jax import lax
from jax.experimental import pallas as pl
from jax.experimental.pallas import tpu as pltpu
from jax.experimental.pallas import tpu_sc as plsc

_LB = 128
_CH = 32


def _sc_gather_rows(table, idx):
    n = idx.shape[0]
    d = table.shape[1]
    info = plsc.get_sparse_core_info()
    nw = info.num_cores * info.num_subcores
    rows_w = n // nw
    nch = rows_w // _CH
    mesh = plsc.VectorSubcoreMesh(core_axis_name="c", subcore_axis_name="s")

    @functools.partial(
        pl.kernel,
        mesh=mesh,
        out_type=jax.ShapeDtypeStruct((n, d), jnp.float32),
        scratch_types=[
            pltpu.VMEM((_CH,), jnp.int32),
            pltpu.VMEM((_CH, d), jnp.float32),
            pltpu.SemaphoreType.DMA,
        ],
    )
    def gather_k(table_hbm, idx_hbm, out_hbm, idx_v, rows_v, sem):
        wid = lax.axis_index("s") * info.num_cores + lax.axis_index("c")
        base = wid * rows_w

        def body(i, carry):
            off = base + i * _CH
            pltpu.sync_copy(idx_hbm.at[pl.ds(off, _CH)], idx_v)
            pltpu.async_copy(table_hbm.at[idx_v], rows_v, sem).wait()
            pltpu.sync_copy(rows_v, out_hbm.at[pl.ds(off, _CH)])
            return carry

        lax.fori_loop(0, nch, body, 0)

    return gather_k(table, idx)


def _tc_assemble(te3, states, mult, state_w, w8, c8):
    b, l, d = te3.shape
    sdim = states.shape[-1]
    spread = w8.shape[0]
    a = spread - 2

    def body(te_ref, st_ref, m_ref, sw_ref, w_ref, c_ref, out_ref):
        te = te_ref[0]
        te2 = te + te
        s_emb = jnp.dot(st_ref[0], sw_ref[...],
                        preferred_element_type=jnp.float32)
        m = m_ref[0]
        out_ref[0, :, 0:d] = m[:, 0][:, None] * w_ref[0][None, :] \
            + c_ref[0][None, :] + te2
        out_ref[0, :, d:2 * d] = s_emb + c_ref[1][None, :] + te2
        for j in range(a):
            k = 2 + j
            out_ref[0, :, k * d:(k + 1) * d] = (
                m[:, k][:, None] * w_ref[k][None, :]
                + c_ref[k][None, :] + te)
        del out_ref

    return pl.pallas_call(
        body,
        grid=(b, l // _LB),
        in_specs=[
            pl.BlockSpec((1, _LB, d), lambda i, j: (i, j, 0)),
            pl.BlockSpec((1, _LB, sdim), lambda i, j: (i, j, 0)),
            pl.BlockSpec((1, _LB, spread), lambda i, j: (i, j, 0)),
            pl.BlockSpec((sdim, d), lambda i, j: (0, 0)),
            pl.BlockSpec((spread, d), lambda i, j: (0, 0)),
            pl.BlockSpec((spread, d), lambda i, j: (0, 0)),
        ],
        out_specs=pl.BlockSpec((1, _LB, spread * d), lambda i, j: (i, j, 0)),
        out_shape=jax.ShapeDtypeStruct((b, l, spread * d), jnp.float32),
        compiler_params=pltpu.CompilerParams(
            dimension_semantics=("parallel", "parallel"),
        ),
    )(te3, states, mult, state_w, w8, c8)


def kernel(states, actions, returns_to_go, time_steps, padding_mask,
           timestep_table, state_W, state_b, return_W, return_b,
           act_W, act_b, action_pos_table):
    b, l, sdim = states.shape
    a = actions.shape[-1]
    d = timestep_table.shape[1]
    spread = 2 + a

    te = _sc_gather_rows(
        timestep_table, time_steps.reshape(b * l).astype(jnp.int32))
    te3 = te.reshape(b, l, d)

    zcol = jnp.zeros((b, l, 1), jnp.float32)
    mult = jnp.concatenate(
        [returns_to_go[..., None], zcol, actions], axis=-1)
    zrow = jnp.zeros((1, d), jnp.float32)
    w8 = jnp.concatenate([return_W[None], zrow] + [act_W[None]] * a, axis=0)
    c8 = jnp.concatenate(
        [return_b[None], state_b[None], act_b[None] + action_pos_table],
        axis=0)

    out = _tc_assemble(te3, states, mult, state_W, w8, c8)
    embeds = out.reshape(b, l * spread, d)
    pm = jnp.repeat(padding_mask, spread, axis=1)
    return embeds, pm


# speedup vs baseline: 2.9709x; 2.2906x over previous
import functools

import jax
import jax.numpy as jnp
from jax import lax
from jax.experimental import pallas as pl
from jax.experimental.pallas import tpu as pltpu
from jax.experimental.pallas import tpu_sc as plsc

_LB = 128
_CH = 32


def _sc_gather_rows(table, idx):
    n = idx.shape[0]
    d = table.shape[1]
    info = plsc.get_sparse_core_info()
    nw = info.num_cores * info.num_subcores
    rows_w = n // nw
    nch = rows_w // _CH
    mesh = plsc.VectorSubcoreMesh(core_axis_name="c", subcore_axis_name="s")

    @functools.partial(
        pl.kernel,
        mesh=mesh,
        out_type=jax.ShapeDtypeStruct((n, d), jnp.float32),
        scratch_types=[
            pltpu.VMEM((_CH,), jnp.int32),
            pltpu.VMEM((_CH, d), jnp.float32),
            pltpu.SemaphoreType.DMA,
        ],
    )
    def gather_k(table_hbm, idx_hbm, out_hbm, idx_v, rows_v, sem):
        wid = lax.axis_index("s") * info.num_cores + lax.axis_index("c")
        base = wid * rows_w

        def body(i, carry):
            off = base + i * _CH
            pltpu.sync_copy(idx_hbm.at[pl.ds(off, _CH)], idx_v)
            pltpu.async_copy(table_hbm.at[idx_v], rows_v, sem).wait()
            pltpu.sync_copy(rows_v, out_hbm.at[pl.ds(off, _CH)])
            return carry

        lax.fori_loop(0, nch, body, 0)

    return gather_k(table, idx)


def _tc_assemble(te3, states, mult, state_w, w8, c8):
    b, l, d = te3.shape
    sdim = states.shape[-1]
    spread = w8.shape[0]
    a = spread - 2

    def body(te_ref, st_ref, m_ref, sw_ref, w_ref, c_ref, out_ref):
        te = te_ref[0]
        te2 = te + te
        s_emb = jnp.dot(st_ref[0], sw_ref[...],
                        preferred_element_type=jnp.float32)
        m = m_ref[0]
        out_ref[0, :, 0, :] = m[:, 0][:, None] * w_ref[0][None, :] \
            + c_ref[0][None, :] + te2
        out_ref[0, :, 1, :] = s_emb + c_ref[1][None, :] + te2
        for j in range(a):
            k = 2 + j
            out_ref[0, :, k, :] = (
                m[:, k][:, None] * w_ref[k][None, :]
                + c_ref[k][None, :] + te)
        del out_ref

    return pl.pallas_call(
        body,
        grid=(b, l // _LB),
        in_specs=[
            pl.BlockSpec((1, _LB, d), lambda i, j: (i, j, 0)),
            pl.BlockSpec((1, _LB, sdim), lambda i, j: (i, j, 0)),
            pl.BlockSpec((1, _LB, spread), lambda i, j: (i, j, 0)),
            pl.BlockSpec((sdim, d), lambda i, j: (0, 0)),
            pl.BlockSpec((spread, d), lambda i, j: (0, 0)),
            pl.BlockSpec((spread, d), lambda i, j: (0, 0)),
        ],
        out_specs=pl.BlockSpec((1, _LB, spread, d), lambda i, j: (i, j, 0, 0)),
        out_shape=jax.ShapeDtypeStruct((b, l, spread, d), jnp.float32),
        compiler_params=pltpu.CompilerParams(
            dimension_semantics=("parallel", "parallel"),
        ),
    )(te3, states, mult, state_w, w8, c8)


def kernel(states, actions, returns_to_go, time_steps, padding_mask,
           timestep_table, state_W, state_b, return_W, return_b,
           act_W, act_b, action_pos_table):
    b, l, sdim = states.shape
    a = actions.shape[-1]
    d = timestep_table.shape[1]
    spread = 2 + a

    te = _sc_gather_rows(
        timestep_table, time_steps.reshape(b * l).astype(jnp.int32))
    te3 = te.reshape(b, l, d)

    zcol = jnp.zeros((b, l, 1), jnp.float32)
    mult = jnp.concatenate(
        [returns_to_go[..., None], zcol, actions], axis=-1)
    zrow = jnp.zeros((1, d), jnp.float32)
    w8 = jnp.concatenate([return_W[None], zrow] + [act_W[None]] * a, axis=0)
    c8 = jnp.concatenate(
        [return_b[None], state_b[None], act_b[None] + action_pos_table],
        axis=0)

    out = _tc_assemble(te3, states, mult, state_W, w8, c8)
    embeds = out.reshape(b, l * spread, d)
    pm = jnp.repeat(padding_mask, spread, axis=1)
    return embeds, pm


# speedup vs baseline: 3.0730x; 1.0344x over previous
import functools

import jax
import jax.numpy as jnp
from jax import lax
from jax.experimental import pallas as pl
from jax.experimental.pallas import tpu as pltpu
from jax.experimental.pallas import tpu_sc as plsc

_LB = 256
_CH = 32


def _sc_gather_rows(table, idx):
    n = idx.shape[0]
    d = table.shape[1]
    info = plsc.get_sparse_core_info()
    nw = info.num_cores * info.num_subcores
    rows_w = n // nw
    nch = rows_w // _CH
    mesh = plsc.VectorSubcoreMesh(core_axis_name="c", subcore_axis_name="s")

    @functools.partial(
        pl.kernel,
        mesh=mesh,
        out_type=jax.ShapeDtypeStruct((n, d), jnp.float32),
        scratch_types=[
            pltpu.VMEM((_CH,), jnp.int32),
            pltpu.VMEM((_CH, d), jnp.float32),
            pltpu.SemaphoreType.DMA,
        ],
    )
    def gather_k(table_hbm, idx_hbm, out_hbm, idx_v, rows_v, sem):
        wid = lax.axis_index("s") * info.num_cores + lax.axis_index("c")
        base = wid * rows_w

        def body(i, carry):
            off = base + i * _CH
            pltpu.sync_copy(idx_hbm.at[pl.ds(off, _CH)], idx_v)
            pltpu.async_copy(table_hbm.at[idx_v], rows_v, sem).wait()
            pltpu.sync_copy(rows_v, out_hbm.at[pl.ds(off, _CH)])
            return carry

        lax.fori_loop(0, nch, body, 0)

    return gather_k(table, idx)


def _tc_assemble(te3, states, mult, state_w, w8, c8):
    b, l, d = te3.shape
    sdim = states.shape[-1]
    spread = w8.shape[0]
    a = spread - 2

    def body(te_ref, st_ref, m_ref, sw_ref, w_ref, c_ref, out_ref):
        te = te_ref[0]
        te2 = te + te
        s_emb = jnp.dot(st_ref[0], sw_ref[...],
                        preferred_element_type=jnp.float32)
        m = m_ref[0]
        out_ref[0, :, 0, :] = m[:, 0][:, None] * w_ref[0][None, :] \
            + c_ref[0][None, :] + te2
        out_ref[0, :, 1, :] = s_emb + c_ref[1][None, :] + te2
        for j in range(a):
            k = 2 + j
            out_ref[0, :, k, :] = (
                m[:, k][:, None] * w_ref[k][None, :]
                + c_ref[k][None, :] + te)
        del out_ref

    return pl.pallas_call(
        body,
        grid=(b, l // _LB),
        in_specs=[
            pl.BlockSpec((1, _LB, d), lambda i, j: (i, j, 0)),
            pl.BlockSpec((1, _LB, sdim), lambda i, j: (i, j, 0)),
            pl.BlockSpec((1, _LB, spread), lambda i, j: (i, j, 0)),
            pl.BlockSpec((sdim, d), lambda i, j: (0, 0)),
            pl.BlockSpec((spread, d), lambda i, j: (0, 0)),
            pl.BlockSpec((spread, d), lambda i, j: (0, 0)),
        ],
        out_specs=pl.BlockSpec((1, _LB, spread, d), lambda i, j: (i, j, 0, 0)),
        out_shape=jax.ShapeDtypeStruct((b, l, spread, d), jnp.float32),
        compiler_params=pltpu.CompilerParams(
            dimension_semantics=("parallel", "parallel"),
        ),
    )(te3, states, mult, state_w, w8, c8)


def kernel(states, actions, returns_to_go, time_steps, padding_mask,
           timestep_table, state_W, state_b, return_W, return_b,
           act_W, act_b, action_pos_table):
    b, l, sdim = states.shape
    a = actions.shape[-1]
    d = timestep_table.shape[1]
    spread = 2 + a

    te = _sc_gather_rows(
        timestep_table, time_steps.reshape(b * l).astype(jnp.int32))
    te3 = te.reshape(b, l, d)

    zcol = jnp.zeros((b, l, 1), jnp.float32)
    mult = jnp.concatenate(
        [returns_to_go[..., None], zcol, actions], axis=-1)
    zrow = jnp.zeros((1, d), jnp.float32)
    w8 = jnp.concatenate([return_W[None], zrow] + [act_W[None]] * a, axis=0)
    c8 = jnp.concatenate(
        [return_b[None], state_b[None], act_b[None] + action_pos_table],
        axis=0)

    out = _tc_assemble(te3, states, mult, state_W, w8, c8)
    embeds = out.reshape(b, l * spread, d)
    pm = jnp.repeat(padding_mask, spread, axis=1)
    return embeds, pm
